# baseline (speedup 1.0000x reference)
import jax
import jax.numpy as jnp
from jax import lax
from jax.experimental import pallas as pl
from jax.experimental.pallas import tpu as pltpu
from jax.experimental.pallas import tpu_sc as plsc

VOCAB = 100000
HIDDEN = 768
BATCH = 64
SEQ = 512
LN_EPS = 1e-5

L = 16
NW = 32
POS_PER_W = SEQ // NW
NSLOT = 4


def _make_sc_gather_body(nb):
    def _sc_gather_body(x_hbm, we_hbm, out_hbm, idx_v, bufs, gsems, osems):
        c = lax.axis_index("c")
        s = lax.axis_index("s")
        wid = s * 2 + c
        s0 = wid * POS_PER_W

        pltpu.sync_copy(x_hbm, idx_v)

        def gather(b, slot):
            return pltpu.make_async_copy(
                we_hbm.at[idx_v.at[b, pl.ds(s0, POS_PER_W)]], bufs[slot],
                gsems[slot])

        def outcopy(b, slot):
            return pltpu.make_async_copy(
                bufs[slot], out_hbm.at[b, pl.ds(s0, POS_PER_W)], osems[slot])

        for slot in range(NSLOT):
            gather(slot, slot).start()

        def b_group(q, _):
            for slot in range(NSLOT):
                b = q * NSLOT + slot
                gather(b, slot).wait()
                outcopy(b, slot).start()
            for slot in range(NSLOT):
                b = q * NSLOT + slot
                outcopy(b, slot).wait()

                @pl.when(b + NSLOT < nb)
                def _refill():
                    gather(b + NSLOT, slot).start()
            return 0

        lax.fori_loop(0, nb // NSLOT, b_group, 0)

    return _sc_gather_body


def _sc_gather(x, word_emb):
    nb = x.shape[0]
    mesh = plsc.VectorSubcoreMesh(core_axis_name="c", subcore_axis_name="s")
    run = pl.kernel(
        _make_sc_gather_body(nb),
        out_type=jax.ShapeDtypeStruct((nb, SEQ, HIDDEN), jnp.float32),
        mesh=mesh,
        scratch_types=[
            pltpu.VMEM((nb, SEQ), jnp.int32),
            [pltpu.VMEM((POS_PER_W, HIDDEN), jnp.float32)
             for _ in range(NSLOT)],
            [pltpu.SemaphoreType.DMA for _ in range(NSLOT)],
            [pltpu.SemaphoreType.DMA for _ in range(NSLOT)],
        ],
    )
    return run(x, word_emb)


BB = 4


def _tc_ln_compute(g_ref, pos_ref, gam_ref, bet_ref, out_ref):
    for bi in range(BB):
        v = g_ref[bi] + pos_ref[...]
        mean = jnp.mean(v, axis=-1, keepdims=True)
        cent = v - mean
        var = jnp.mean(cent * cent, axis=-1, keepdims=True)
        normed = cent * lax.rsqrt(var + LN_EPS)
        out_ref[bi] = normed * gam_ref[0] + bet_ref[0]


def _tc_ln_body_first(g_ref, pos_ref, gam_ref, bet_ref, out_ref):
    _tc_ln_compute(g_ref, pos_ref, gam_ref, bet_ref, out_ref)


def _tc_ln_body_chain(prev_ref, g_ref, pos_ref, gam_ref, bet_ref, out_ref):
    del prev_ref
    _tc_ln_compute(g_ref, pos_ref, gam_ref, bet_ref, out_ref)


def _tc_ln_chunk(prev, g, pos_emb, gamma2d, beta2d, b0):
    nb = g.shape[0]
    bblk0 = b0 // BB
    small_specs = [
        pl.BlockSpec((SEQ, HIDDEN), lambda i: (0, 0)),
        pl.BlockSpec((1, HIDDEN), lambda i: (0, 0)),
        pl.BlockSpec((1, HIDDEN), lambda i: (0, 0)),
    ]
    out_spec = pl.BlockSpec((BB, SEQ, HIDDEN), lambda i: (bblk0 + i, 0, 0))
    out_shape = jax.ShapeDtypeStruct((BATCH, SEQ, HIDDEN), jnp.float32)
    if prev is None:
        return pl.pallas_call(
            _tc_ln_body_first,
            grid=(nb // BB,),
            in_specs=[pl.BlockSpec((BB, SEQ, HIDDEN), lambda i: (i, 0, 0))]
            + small_specs,
            out_specs=out_spec,
            out_shape=out_shape,
        )(g, pos_emb, gamma2d, beta2d)
    return pl.pallas_call(
        _tc_ln_body_chain,
        grid=(nb // BB,),
        in_specs=[pl.BlockSpec(memory_space=pl.ANY),
                  pl.BlockSpec((BB, SEQ, HIDDEN), lambda i: (i, 0, 0))]
        + small_specs,
        out_specs=out_spec,
        out_shape=out_shape,
        input_output_aliases={0: 0},
    )(prev, g, pos_emb, gamma2d, beta2d)


CHUNKS = (16, 16, 16, 16)


@jax.jit
def kernel(x, word_emb, pos_emb, ln_gamma, ln_beta):
    gamma2d = ln_gamma.reshape(1, HIDDEN)
    beta2d = ln_beta.reshape(1, HIDDEN)
    starts = [sum(CHUNKS[:c]) for c in range(len(CHUNKS))]
    gs = [_sc_gather(x[b0:b0 + nb], word_emb)
          for b0, nb in zip(starts, CHUNKS)]
    out = None
    for c, (b0, nb) in enumerate(zip(starts, CHUNKS)):
        out = _tc_ln_chunk(out, gs[c], pos_emb, gamma2d, beta2d, b0)
    return out


# speedup vs baseline: 1.0690x; 1.0690x over previous
import jax
import jax.numpy as jnp
from jax import lax
from jax.experimental import pallas as pl
from jax.experimental.pallas import tpu as pltpu
from jax.experimental.pallas import tpu_sc as plsc

VOCAB = 100000
HIDDEN = 768
BATCH = 64
SEQ = 512
LN_EPS = 1e-5

L = 16
NW = 32
POS_PER_W = SEQ // NW
NSLOT = 4


def _make_sc_gather_body(nb):
    def _sc_gather_body(x_hbm, we_hbm, out_hbm, idx_v, bufs, gsems, osems):
        c = lax.axis_index("c")
        s = lax.axis_index("s")
        wid = s * 2 + c
        s0 = wid * POS_PER_W

        pltpu.sync_copy(x_hbm, idx_v)

        def gather(b, slot):
            return pltpu.make_async_copy(
                we_hbm.at[idx_v.at[b, pl.ds(s0, POS_PER_W)]], bufs[slot],
                gsems[slot])

        def outcopy(b, slot):
            return pltpu.make_async_copy(
                bufs[slot], out_hbm.at[b, pl.ds(s0, POS_PER_W)], osems[slot])

        for slot in range(NSLOT):
            gather(slot, slot).start()

        def b_group(q, _):
            for slot in range(NSLOT):
                b = q * NSLOT + slot
                gather(b, slot).wait()
                outcopy(b, slot).start()
            for slot in range(NSLOT):
                b = q * NSLOT + slot
                outcopy(b, slot).wait()

                @pl.when(b + NSLOT < nb)
                def _refill():
                    gather(b + NSLOT, slot).start()
            return 0

        lax.fori_loop(0, nb // NSLOT, b_group, 0)

    return _sc_gather_body


def _sc_gather(x, word_emb):
    nb = x.shape[0]
    mesh = plsc.VectorSubcoreMesh(core_axis_name="c", subcore_axis_name="s")
    run = pl.kernel(
        _make_sc_gather_body(nb),
        out_type=jax.ShapeDtypeStruct((nb, SEQ, HIDDEN), jnp.float32),
        mesh=mesh,
        scratch_types=[
            pltpu.VMEM((nb, SEQ), jnp.int32),
            [pltpu.VMEM((POS_PER_W, HIDDEN), jnp.float32)
             for _ in range(NSLOT)],
            [pltpu.SemaphoreType.DMA for _ in range(NSLOT)],
            [pltpu.SemaphoreType.DMA for _ in range(NSLOT)],
        ],
    )
    return run(x, word_emb)


BB = 4


def _tc_ln_compute(g_ref, pos_ref, gam_ref, bet_ref, out_ref):
    for bi in range(BB):
        v = g_ref[bi] + pos_ref[...]
        mean = jnp.mean(v, axis=-1, keepdims=True)
        cent = v - mean
        var = jnp.mean(cent * cent, axis=-1, keepdims=True)
        normed = cent * lax.rsqrt(var + LN_EPS)
        out_ref[bi] = normed * gam_ref[0] + bet_ref[0]


def _tc_ln_body_first(g_ref, pos_ref, gam_ref, bet_ref, out_ref):
    _tc_ln_compute(g_ref, pos_ref, gam_ref, bet_ref, out_ref)


def _tc_ln_body_chain(prev_ref, g_ref, pos_ref, gam_ref, bet_ref, out_ref):
    del prev_ref
    _tc_ln_compute(g_ref, pos_ref, gam_ref, bet_ref, out_ref)


def _tc_ln_chunk(prev, g, pos_emb, gamma2d, beta2d, b0):
    nb = g.shape[0]
    bblk0 = b0 // BB
    small_specs = [
        pl.BlockSpec((SEQ, HIDDEN), lambda i: (0, 0)),
        pl.BlockSpec((1, HIDDEN), lambda i: (0, 0)),
        pl.BlockSpec((1, HIDDEN), lambda i: (0, 0)),
    ]
    out_spec = pl.BlockSpec((BB, SEQ, HIDDEN), lambda i: (bblk0 + i, 0, 0))
    out_shape = jax.ShapeDtypeStruct((BATCH, SEQ, HIDDEN), jnp.float32)
    if prev is None:
        return pl.pallas_call(
            _tc_ln_body_first,
            grid=(nb // BB,),
            in_specs=[pl.BlockSpec((BB, SEQ, HIDDEN), lambda i: (i, 0, 0))]
            + small_specs,
            out_specs=out_spec,
            out_shape=out_shape,
        )(g, pos_emb, gamma2d, beta2d)
    return pl.pallas_call(
        _tc_ln_body_chain,
        grid=(nb // BB,),
        in_specs=[pl.BlockSpec(memory_space=pl.ANY),
                  pl.BlockSpec((BB, SEQ, HIDDEN), lambda i: (i, 0, 0))]
        + small_specs,
        out_specs=out_spec,
        out_shape=out_shape,
        input_output_aliases={0: 0},
    )(prev, g, pos_emb, gamma2d, beta2d)


CHUNKS = (32, 32)


@jax.jit
def kernel(x, word_emb, pos_emb, ln_gamma, ln_beta):
    gamma2d = ln_gamma.reshape(1, HIDDEN)
    beta2d = ln_beta.reshape(1, HIDDEN)
    starts = [sum(CHUNKS[:c]) for c in range(len(CHUNKS))]
    gs = [_sc_gather(x[b0:b0 + nb], word_emb)
          for b0, nb in zip(starts, CHUNKS)]
    out = None
    for c, (b0, nb) in enumerate(zip(starts, CHUNKS)):
        out = _tc_ln_chunk(out, gs[c], pos_emb, gamma2d, beta2d, b0)
    return out
